# Initial kernel scaffold; baseline (speedup 1.0000x reference)
import jax
import jax.numpy as jnp
from jax.experimental import pallas as pl


def kernel(x, edge_index0, edge_index1, edge_index2, W0l, W0r, b0, W1l, W1r, b1, W2l, W2r, b2, W3l, W3r, b3):
    raise NotImplementedError("write your pallas kernel here")


# speedup vs baseline: 4.0658x; 4.0658x over previous
import functools

import jax
import jax.numpy as jnp
from jax import lax
from jax.experimental import pallas as pl
from jax.experimental.pallas import tpu as pltpu
from jax.experimental.pallas import tpu_sc as plsc

_N1, _N2, _N3 = 20000, 5000, 1024
_E0, _E1, _E2 = 320000, 80000, 16384

_CH = 128
_NS = 16
_ZR = 128


def _make_agg(E, N, Dh):
  assert E % _CH == 0
  npad = ((N + _NS * 8 - 1) // (_NS * 8)) * (_NS * 8)
  rpt = npad // _NS
  chunks = E // _CH
  nf, tl = divmod(rpt, _ZR)

  mesh = plsc.VectorSubcoreMesh(core_axis_name="c", subcore_axis_name="s")
  out_type = (
      jax.ShapeDtypeStruct((npad, Dh), jnp.float32),
      jax.ShapeDtypeStruct((npad, Dh), jnp.float32),
      jax.ShapeDtypeStruct((npad, 16), jnp.float32),
  )

  @functools.partial(
      pl.kernel,
      out_type=out_type,
      mesh=mesh,
      compiler_params=pltpu.CompilerParams(use_tc_tiling_on_sc=False),
      scratch_types=[
          pltpu.VMEM((_CH,), jnp.int32),
          pltpu.VMEM((_CH,), jnp.int32),
          pltpu.VMEM((_CH, Dh), jnp.float32),
          pltpu.VMEM((_CH, 16), jnp.float32),
          pltpu.VMEM((_ZR, Dh), jnp.float32),
          pltpu.VMEM((_ZR, 16), jnp.float32),
          pltpu.VMEM_SHARED((npad, Dh), jnp.float32),
          pltpu.VMEM_SHARED((npad, 16), jnp.float32),
          pltpu.SemaphoreType.DMA,
      ],
  )
  def agg(t0, t1, src, dst, out0, out1, cnt_out,
          src_v, dst_v, rows_v, ones_v, zf, z16, acc, cnt, sem):
    c = lax.axis_index("c")
    s = lax.axis_index("s")

    zv = jnp.zeros((16,), jnp.float32)
    ov = jnp.ones((16,), jnp.float32)

    def fill_row(i, _):
      for j in range(Dh // 16):
        zf[i, pl.ds(j * 16, 16)] = zv
      z16[i, pl.ds(0, 16)] = zv
      ones_v[i, pl.ds(0, 16)] = ov
      return 0

    lax.fori_loop(0, _ZR, fill_row, 0)

    r0 = pl.multiple_of(s * rpt, 8)
    for f in range(nf):
      pltpu.sync_copy(zf, acc.at[pl.ds(r0 + f * _ZR, _ZR)])
    if tl:
      pltpu.sync_copy(zf.at[pl.ds(0, tl)], acc.at[pl.ds(r0 + nf * _ZR, tl)])

    @pl.when(c == 0)
    def _():
      for f in range(nf):
        pltpu.sync_copy(z16, cnt.at[pl.ds(r0 + f * _ZR, _ZR)])
      if tl:
        pltpu.sync_copy(z16.at[pl.ds(0, tl)], cnt.at[pl.ds(r0 + nf * _ZR, tl)])

    plsc.subcore_barrier()

    njs = (chunks - s + _NS - 1) // _NS

    def run(tab, with_cnt):
      def body(j, _):
        base = (s + j * _NS) * _CH
        pltpu.sync_copy(src.at[pl.ds(base, _CH)], src_v)
        pltpu.sync_copy(dst.at[pl.ds(base, _CH)], dst_v)
        pltpu.async_copy(tab.at[src_v], rows_v, sem).wait()
        pltpu.sync_copy(rows_v, acc.at[dst_v], add=True)
        if with_cnt:
          pltpu.sync_copy(ones_v, cnt.at[dst_v], add=True)
        return 0

      lax.fori_loop(0, njs, body, 0)

    @pl.when(c == 0)
    def _():
      run(t0, True)

    @pl.when(c == 1)
    def _():
      run(t1, False)

    plsc.subcore_barrier()

    @pl.when(c == 0)
    def _():
      pltpu.sync_copy(acc.at[pl.ds(r0, rpt)], out0.at[pl.ds(r0, rpt)])
      pltpu.sync_copy(cnt.at[pl.ds(r0, rpt)], cnt_out.at[pl.ds(r0, rpt)])

    @pl.when(c == 1)
    def _():
      pltpu.sync_copy(acc.at[pl.ds(r0, rpt)], out1.at[pl.ds(r0, rpt)])

  return agg


def _make_mm(M, TM, Dh, Dx, Do, relu):
  assert M % TM == 0
  grid = (M // TM,)

  def body(s0, s1, cnt, xt, wl, wr, b, out):
    inv = 1.0 / jnp.maximum(cnt[:, 0:1], 1.0)
    a0 = s0[...] * inv
    a1 = s1[...] * inv
    acc = jnp.dot(a0, wl[0:Dh, :], preferred_element_type=jnp.float32)
    acc = acc + jnp.dot(a1, wl[Dh:2 * Dh, :], preferred_element_type=jnp.float32)
    acc = acc + jnp.dot(xt[...], wr[...], preferred_element_type=jnp.float32)
    acc = acc + b[...]
    out[...] = jnp.maximum(acc, 0.0) if relu else acc

  return pl.pallas_call(
      body,
      grid=grid,
      in_specs=[
          pl.BlockSpec((TM, Dh), lambda i: (i, 0)),
          pl.BlockSpec((TM, Dh), lambda i: (i, 0)),
          pl.BlockSpec((TM, 16), lambda i: (i, 0)),
          pl.BlockSpec((TM, Dx), lambda i: (i, 0)),
          pl.BlockSpec((2 * Dh, Do), lambda i: (0, 0)),
          pl.BlockSpec((Dx, Do), lambda i: (0, 0)),
          pl.BlockSpec((1, Do), lambda i: (0, 0)),
      ],
      out_specs=pl.BlockSpec((TM, Do), lambda i: (i, 0)),
      out_shape=jax.ShapeDtypeStruct((M, Do), jnp.float32),
  )


_agg0 = _make_agg(_E0, _N1, 64)
_agg1 = _make_agg(_E1, _N2, 128)
_agg2 = _make_agg(_E2, _N3, 128)
_mm0 = _make_mm(_N1, 400, 64, 128, 256, True)
_mm1 = _make_mm(_N2, 200, 128, 256, 256, True)
_mm2 = _make_mm(_N3, 256, 128, 256, 256, False)


@jax.jit
def kernel(x, edge_index0, edge_index1, edge_index2,
           W0l, W0r, b0, W1l, W1r, b1, W2l, W2r, b2, W3l, W3r, b3):
  x0 = x[:_N1, :64]
  x1 = x[:_N1, 64:]
  s0a, s0b, c0 = _agg0(x0, x1, edge_index0[0], edge_index0[1])
  h0 = _mm0(s0a, s0b, c0, x, W0l, W0r, b0.reshape(1, -1))

  t1a = h0[:_N2, :128]
  t1b = h0[:_N2, 128:]
  s1a, s1b, c1 = _agg1(t1a, t1b, edge_index1[0], edge_index1[1])
  h1 = _mm1(s1a, s1b, c1, h0, W1l, W1r, b1.reshape(1, -1))

  t2a = h1[:_N3, :128]
  t2b = h1[:_N3, 128:]
  s2a, s2b, c2 = _agg2(t2a, t2b, edge_index2[0], edge_index2[1])
  wl = jnp.concatenate([W2l, W3l], axis=1)
  wr = jnp.concatenate([W2r, W3r], axis=1)
  bb = jnp.concatenate([b2, b3]).reshape(1, -1)
  out = _mm2(s2a, s2b, c2, h1, wl, wr, bb)
  return out[:, :128], out[:, 128:]
